# baseline (speedup 1.0000x reference)
import functools

import jax
import jax.numpy as jnp
from jax import lax
from jax.experimental import pallas as pl
from jax.experimental.pallas import tpu as pltpu
from jax.experimental.pallas import tpu_sc as plsc

B, S = 4, 8192
F = 1024
Q = 256
N = B * S
NC, NS = 2, 16
NW = NC * NS
RPW = N // NW
CH = 32
NCH = RPW // CH
L = 16


def _body(table, types3, pos_all, ids_all, hsn_all, lay,
          out,
          stg0, stg1, emb0, emb1, hsnbuf, types_v, pos_v, ids_v, lay_v,
          sg0, sg1, so0, so1):
    wid = lax.axis_index("s") * NC + lax.axis_index("c")
    row0w = wid * RPW
    iota = lax.iota(jnp.int32, L)
    z16 = jnp.zeros((L,), jnp.float32)

    pltpu.sync_copy(hsn_all, hsnbuf)

    def _mx(i, acc):
        return jnp.maximum(acc, hsnbuf[pl.ds(i * L, L)])

    maxv = jnp.max(lax.fori_loop(0, N // L, _mx, jnp.full((L,), -1e30, jnp.float32)))

    pltpu.sync_copy(lay, lay_v)
    layv = lay_v[...]

    def _zrow(r, carry):
        for k in range(Q, F, L):
            stg0[r, pl.ds(k, L)] = z16
            stg1[r, pl.ds(k, L)] = z16
        return carry

    lax.fori_loop(0, CH, _zrow, 0)
    for stg in (stg0, stg1):
        for g in range(CH // L):
            rows = g * L + iota
            plsc.store_scatter(stg, [rows, jnp.full((L,), Q + 3, jnp.int32)], layv)

    pltpu.sync_copy(types3.at[wid], types_v)
    pltpu.sync_copy(pos_all.at[pl.ds(row0w, RPW)], pos_v)
    pltpu.sync_copy(ids_all.at[pl.ds(row0w, RPW)], ids_v)

    stgs = (stg0, stg1)
    embs = (emb0, emb1)
    sgs = (sg0, sg1)
    sos = (so0, so1)
    outstanding = [None, None]

    for c in range(NCH):
        b = c % 2
        stg, emb, sg, so = stgs[b], embs[b], sgs[b], sos[b]
        if outstanding[b] is not None:
            outstanding[b].wait()
        gh = pltpu.async_copy(table.at[types_v.at[c]], emb, sg)
        for g in range(CH // L):
            off = c * CH + g * L
            rows = g * L + iota
            posv = pos_v[pl.ds(off, L)].astype(jnp.float32) * (1.0 / S)
            specv = jnp.where(ids_v[pl.ds(off, L)] < 5, 1.0, 0.0).astype(jnp.float32)
            hv = hsnbuf[pl.ds(row0w + off, L)] / maxv
            plsc.store_scatter(stg, [rows, jnp.full((L,), Q + 0, jnp.int32)], posv)
            plsc.store_scatter(stg, [rows, jnp.full((L,), Q + 1, jnp.int32)], specv)
            plsc.store_scatter(stg, [rows, jnp.full((L,), Q + 2, jnp.int32)], hv)
        gh.wait()

        def _cprow(r, carry, stg=stg, emb=emb):
            for k in range(0, Q, L):
                stg[r, pl.ds(k, L)] = emb[r, pl.ds(k, L)]
            return carry

        lax.fori_loop(0, CH, _cprow, 0)
        outstanding[b] = pltpu.async_copy(
            stg, out.at[pl.ds(row0w + c * CH, CH)], so
        )

    for b in range(2):
        if outstanding[b] is not None:
            outstanding[b].wait()


@jax.jit
def _run(table, types3, pos_all, ids_all, hsn_all, lay):
    mesh = plsc.VectorSubcoreMesh(
        core_axis_name="c", subcore_axis_name="s", num_cores=NC, num_subcores=NS
    )
    f = functools.partial(
        pl.kernel,
        out_type=jax.ShapeDtypeStruct((N, F), jnp.float32),
        mesh=mesh,
        scratch_types=[
            pltpu.VMEM((CH, F), jnp.float32),
            pltpu.VMEM((CH, F), jnp.float32),
            pltpu.VMEM((CH, Q), jnp.float32),
            pltpu.VMEM((CH, Q), jnp.float32),
            pltpu.VMEM((N,), jnp.float32),
            pltpu.VMEM((NCH, CH), jnp.int32),
            pltpu.VMEM((RPW,), jnp.int32),
            pltpu.VMEM((RPW,), jnp.int32),
            pltpu.VMEM((L,), jnp.float32),
            pltpu.SemaphoreType.DMA,
            pltpu.SemaphoreType.DMA,
            pltpu.SemaphoreType.DMA,
            pltpu.SemaphoreType.DMA,
        ],
        compiler_params=pltpu.CompilerParams(
            use_tc_tiling_on_sc=False, needs_layout_passes=False
        ),
    )(_body)
    return f(table, types3, pos_all, ids_all, hsn_all, lay)


def kernel(input_ids, token_type_ids, positions, hidden_state_norms,
           layer_idx, token_type_table):
    types3 = token_type_ids.reshape(NW, NCH, CH)
    pos_all = positions.reshape(N)
    ids_all = input_ids.reshape(N)
    hsn_all = hidden_state_norms.reshape(N)
    lay = jnp.zeros((L,), jnp.float32) + jnp.asarray(layer_idx, jnp.float32) / 100.0
    out = _run(token_type_table, types3, pos_all, ids_all, hsn_all, lay)
    return out.reshape(B, S, F)


# speedup vs baseline: 1.0009x; 1.0009x over previous
import functools

import jax
import jax.numpy as jnp
from jax import lax
from jax.experimental import pallas as pl
from jax.experimental.pallas import tpu as pltpu
from jax.experimental.pallas import tpu_sc as plsc

B, S = 4, 8192
F = 1024
Q = 256
N = B * S
NC, NS = 2, 16
NW = NC * NS
RPW = N // NW
CH = 32
NCH = RPW // CH
L = 16


def _body(table, types3, pos_all, ids_all, hsn_all, lay,
          out,
          stg0, stg1, emb0, emb1, hsnbuf, types_v, pos_v, ids_v, lay_v,
          sg0, sg1, so0, so1):
    wid = lax.axis_index("s") * NC + lax.axis_index("c")
    row0w = wid * RPW
    bidx = row0w // S
    srow0 = row0w % S
    iota = lax.iota(jnp.int32, L)
    z16 = jnp.zeros((L,), jnp.float32)

    pltpu.sync_copy(hsn_all, hsnbuf)

    def _mx(i, acc):
        return jnp.maximum(acc, hsnbuf[pl.ds(i * L, L)])

    maxv = jnp.max(lax.fori_loop(0, N // L, _mx, jnp.full((L,), -1e30, jnp.float32)))

    pltpu.sync_copy(lay, lay_v)
    layv = lay_v[...]

    def _zrow(r, carry):
        for k in range(Q, F, L):
            stg0[r, pl.ds(k, L)] = z16
            stg1[r, pl.ds(k, L)] = z16
        return carry

    lax.fori_loop(0, CH, _zrow, 0)
    for stg in (stg0, stg1):
        for g in range(CH // L):
            rows = g * L + iota
            plsc.store_scatter(stg, [rows, jnp.full((L,), Q + 3, jnp.int32)], layv)

    pltpu.sync_copy(types3.at[wid], types_v)
    pltpu.sync_copy(pos_all.at[pl.ds(row0w, RPW)], pos_v)
    pltpu.sync_copy(ids_all.at[pl.ds(row0w, RPW)], ids_v)

    stgs = (stg0, stg1)
    embs = (emb0, emb1)
    sgs = (sg0, sg1)
    sos = (so0, so1)
    outstanding = [None, None]

    for c in range(NCH):
        b = c % 2
        stg, emb, sg, so = stgs[b], embs[b], sgs[b], sos[b]
        if outstanding[b] is not None:
            outstanding[b].wait()
        gh = pltpu.async_copy(table.at[types_v.at[c]], emb, sg)
        for g in range(CH // L):
            off = c * CH + g * L
            rows = g * L + iota
            posv = pos_v[pl.ds(off, L)].astype(jnp.float32) * (1.0 / S)
            specv = jnp.where(ids_v[pl.ds(off, L)] < 5, 1.0, 0.0).astype(jnp.float32)
            hv = hsnbuf[pl.ds(row0w + off, L)] / maxv
            plsc.store_scatter(stg, [rows, jnp.full((L,), Q + 0, jnp.int32)], posv)
            plsc.store_scatter(stg, [rows, jnp.full((L,), Q + 1, jnp.int32)], specv)
            plsc.store_scatter(stg, [rows, jnp.full((L,), Q + 2, jnp.int32)], hv)
        gh.wait()

        def _cprow(r, carry, stg=stg, emb=emb):
            for k in range(0, Q, L):
                stg[r, pl.ds(k, L)] = emb[r, pl.ds(k, L)]
            return carry

        lax.fori_loop(0, CH, _cprow, 0)
        outstanding[b] = pltpu.async_copy(
            stg, out.at[bidx].at[pl.ds(srow0 + c * CH, CH)], so
        )

    for b in range(2):
        if outstanding[b] is not None:
            outstanding[b].wait()


@jax.jit
def _run(table, types3, pos_all, ids_all, hsn_all, lay):
    mesh = plsc.VectorSubcoreMesh(
        core_axis_name="c", subcore_axis_name="s", num_cores=NC, num_subcores=NS
    )
    f = functools.partial(
        pl.kernel,
        out_type=jax.ShapeDtypeStruct((B, S, F), jnp.float32),
        mesh=mesh,
        scratch_types=[
            pltpu.VMEM((CH, F), jnp.float32),
            pltpu.VMEM((CH, F), jnp.float32),
            pltpu.VMEM((CH, Q), jnp.float32),
            pltpu.VMEM((CH, Q), jnp.float32),
            pltpu.VMEM((N,), jnp.float32),
            pltpu.VMEM((NCH, CH), jnp.int32),
            pltpu.VMEM((RPW,), jnp.int32),
            pltpu.VMEM((RPW,), jnp.int32),
            pltpu.VMEM((L,), jnp.float32),
            pltpu.SemaphoreType.DMA,
            pltpu.SemaphoreType.DMA,
            pltpu.SemaphoreType.DMA,
            pltpu.SemaphoreType.DMA,
        ],
        compiler_params=pltpu.CompilerParams(
            use_tc_tiling_on_sc=False, needs_layout_passes=False
        ),
    )(_body)
    return f(table, types3, pos_all, ids_all, hsn_all, lay)


def kernel(input_ids, token_type_ids, positions, hidden_state_norms,
           layer_idx, token_type_table):
    types3 = token_type_ids.reshape(NW, NCH, CH)
    pos_all = positions.reshape(N)
    ids_all = input_ids.reshape(N)
    hsn_all = hidden_state_norms.reshape(N)
    lay = jnp.zeros((L,), jnp.float32) + jnp.asarray(layer_idx, jnp.float32) / 100.0
    return _run(token_type_table, types3, pos_all, ids_all, hsn_all, lay)


# speedup vs baseline: 1.4007x; 1.3994x over previous
import functools

import jax
import jax.numpy as jnp
from jax import lax
from jax.experimental import pallas as pl
from jax.experimental.pallas import tpu as pltpu
from jax.experimental.pallas import tpu_sc as plsc

B, S = 4, 8192
F = 1024
Q = 256
N = B * S
NC, NS = 2, 16
NW = NC * NS
RPW = N // NW
CH = 32
NCH = RPW // CH
L = 16


def _body(table, types3, pos_all, ids_all, hsn_all, lay,
          out,
          stg0, stg1, emb0, emb1, hsnbuf, types_v, pos_v, ids_v, lay_v,
          sg0, sg1, so0, so1):
    wid = lax.axis_index("s") * NC + lax.axis_index("c")
    row0w = wid * RPW
    bidx = row0w // S
    srow0 = row0w % S
    iota = lax.iota(jnp.int32, L)
    z16 = jnp.zeros((L,), jnp.float32)

    pltpu.sync_copy(hsn_all, hsnbuf)

    def _mx(i, acc):
        return jnp.maximum(acc, hsnbuf[pl.ds(i * L, L)])

    maxv = jnp.max(lax.fori_loop(0, N // L, _mx, jnp.full((L,), -1e30, jnp.float32)))

    pltpu.sync_copy(lay, lay_v)
    layv = lay_v[...]

    def _zrow(r, carry):
        for k in range(Q, F, L):
            stg0[r, pl.ds(k, L)] = z16
            stg1[r, pl.ds(k, L)] = z16
        return carry

    lax.fori_loop(0, CH, _zrow, 0)
    for stg in (stg0, stg1):
        for g in range(CH // L):
            rows = g * L + iota
            plsc.store_scatter(stg, [rows, jnp.full((L,), Q + 3, jnp.int32)], layv)

    pltpu.sync_copy(types3.at[wid], types_v)
    pltpu.sync_copy(pos_all.at[pl.ds(row0w, RPW)], pos_v)
    pltpu.sync_copy(ids_all.at[pl.ds(row0w, RPW)], ids_v)

    stgs = (stg0, stg1)
    embs = (emb0, emb1)
    sgs = (sg0, sg1)
    sos = (so0, so1)
    outstanding = [None, None]

    for c in range(NCH):
        b = c % 2
        stg, emb, sg, so = stgs[b], embs[b], sgs[b], sos[b]
        if outstanding[b] is not None:
            outstanding[b].wait()
        gh = pltpu.async_copy(table.at[types_v.at[c]], emb, sg)
        for g in range(CH // L):
            off = c * CH + g * L
            rows = g * L + iota
            posv = pos_v[pl.ds(off, L)].astype(jnp.float32) * (1.0 / S)
            specv = jnp.where(ids_v[pl.ds(off, L)] < 5, 1.0, 0.0).astype(jnp.float32)
            hv = hsnbuf[pl.ds(row0w + off, L)] / maxv
            plsc.store_scatter(stg, [rows, jnp.full((L,), Q + 0, jnp.int32)], posv)
            plsc.store_scatter(stg, [rows, jnp.full((L,), Q + 1, jnp.int32)], specv)
            plsc.store_scatter(stg, [rows, jnp.full((L,), Q + 2, jnp.int32)], hv)
        gh.wait()

        def _cprow(r, carry, stg=stg, emb=emb):
            for k in range(0, Q, L):
                stg[r, pl.ds(k, L)] = emb[r, pl.ds(k, L)]
            return carry

        lax.fori_loop(0, CH, _cprow, 0)
        outstanding[b] = pltpu.async_copy(
            stg, out.at[bidx].at[pl.ds(srow0 + c * CH, CH)], so
        )

    for b in range(2):
        if outstanding[b] is not None:
            outstanding[b].wait()


@jax.jit
def _run(table, types3, pos_all, ids_all, hsn_all, lay):
    mesh = plsc.VectorSubcoreMesh(
        core_axis_name="c", subcore_axis_name="s", num_cores=NC, num_subcores=NS
    )
    f = functools.partial(
        pl.kernel,
        out_type=jax.ShapeDtypeStruct((B, S, F), jnp.float32),
        mesh=mesh,
        scratch_types=[
            pltpu.VMEM((CH, F), jnp.float32),
            pltpu.VMEM((CH, F), jnp.float32),
            pltpu.VMEM((CH, Q), jnp.float32),
            pltpu.VMEM((CH, Q), jnp.float32),
            pltpu.VMEM((N,), jnp.float32),
            pltpu.VMEM((NCH, CH), jnp.int32),
            pltpu.VMEM((RPW,), jnp.int32),
            pltpu.VMEM((RPW,), jnp.int32),
            pltpu.VMEM((L,), jnp.float32),
            pltpu.SemaphoreType.DMA,
            pltpu.SemaphoreType.DMA,
            pltpu.SemaphoreType.DMA,
            pltpu.SemaphoreType.DMA,
        ],
        compiler_params=pltpu.CompilerParams(
            use_tc_tiling_on_sc=True, needs_layout_passes=False
        ),
    )(_body)
    return f(table, types3, pos_all, ids_all, hsn_all, lay)


def kernel(input_ids, token_type_ids, positions, hidden_state_norms,
           layer_idx, token_type_table):
    types3 = token_type_ids.reshape(NW, NCH, CH)
    pos_all = positions.reshape(N)
    ids_all = input_ids.reshape(N)
    hsn_all = hidden_state_norms.reshape(N)
    lay = jnp.zeros((L,), jnp.float32) + jnp.asarray(layer_idx, jnp.float32) / 100.0
    return _run(token_type_table, types3, pos_all, ids_all, hsn_all, lay)


# speedup vs baseline: 3.8024x; 2.7147x over previous
import functools

import jax
import jax.numpy as jnp
from jax import lax
from jax.experimental import pallas as pl
from jax.experimental.pallas import tpu as pltpu
from jax.experimental.pallas import tpu_sc as plsc

B, S = 4, 8192
F = 1024
Q = 256
T = 10
N = B * S
NC, NS = 2, 16
NW = NC * NS
RPW = N // NW
CH = 32
NCH = RPW // CH
L = 16
NBUF = 3


def _body(table, types_all, pos_all, ids_all, hsn_all, hsn2d, lay,
          out,
          stg0, stg1, stg2, tloc, typesv, posi, idsi, posf, specf, hsnf, lay_v,
          so0, so1, so2):
    wid = lax.axis_index("s") * NC + lax.axis_index("c")
    row0w = wid * RPW
    bidx = row0w // S
    srow0 = row0w % S
    iota = lax.iota(jnp.int32, L)
    z16 = jnp.zeros((L,), jnp.float32)
    stgs = (stg0, stg1, stg2)
    sos = (so0, so1, so2)

    pltpu.sync_copy(hsn2d, stg0)

    def _mxrow(r, acc):
        for j in range(F // L):
            acc = jnp.maximum(acc, stg0[r, pl.ds(j * L, L)])
        return acc

    maxv = jnp.max(lax.fori_loop(0, CH, _mxrow, jnp.full((L,), -1.0, jnp.float32)))

    pltpu.sync_copy(table, tloc)
    pltpu.sync_copy(types_all.at[pl.ds(row0w, RPW)], typesv)
    pltpu.sync_copy(pos_all.at[pl.ds(row0w, RPW)], posi)
    pltpu.sync_copy(ids_all.at[pl.ds(row0w, RPW)], idsi)
    pltpu.sync_copy(hsn_all.at[pl.ds(row0w, RPW)], hsnf)
    pltpu.sync_copy(lay, lay_v)
    layv = lay_v[...]

    def _cols(j, carry):
        sl = pl.ds(j * L, L)
        posf[sl] = posi[sl].astype(jnp.float32) * (1.0 / S)
        specf[sl] = jnp.where(idsi[sl] < 5, 1.0, 0.0).astype(jnp.float32)
        hsnf[sl] = hsnf[sl] / maxv
        return carry

    lax.fori_loop(0, RPW // L, _cols, 0)

    def _zrow(r, carry):
        for stg in stgs:
            for k in range(Q, F, L):
                stg[r, pl.ds(k, L)] = z16
        return carry

    lax.fori_loop(0, CH, _zrow, 0)
    colL = jnp.full((L,), Q + 3, jnp.int32)
    for stg in stgs:
        for g in range(CH // L):
            rows = g * L + iota
            plsc.store_scatter(stg, [rows, colL], layv)

    cols = [iota + k * L for k in range(Q // L)]
    col0 = jnp.full((L,), Q + 0, jnp.int32)
    col1 = jnp.full((L,), Q + 1, jnp.int32)
    col2 = jnp.full((L,), Q + 2, jnp.int32)
    outstanding = [None] * NBUF

    for c in range(NCH):
        b = c % NBUF
        stg, so = stgs[b], sos[b]
        if outstanding[b] is not None:
            outstanding[b].wait()

        def _erow(r, carry, stg=stg, c=c):
            tfull = plsc.load_gather(typesv, [jnp.zeros((L,), jnp.int32) + (c * CH + r)])
            for k in range(Q // L):
                stg[r, pl.ds(k * L, L)] = plsc.load_gather(tloc, [tfull, cols[k]])
            return carry

        lax.fori_loop(0, CH, _erow, 0)

        for g in range(CH // L):
            off = c * CH + g * L
            rows = g * L + iota
            plsc.store_scatter(stg, [rows, col0], posf[pl.ds(off, L)])
            plsc.store_scatter(stg, [rows, col1], specf[pl.ds(off, L)])
            plsc.store_scatter(stg, [rows, col2], hsnf[pl.ds(off, L)])

        outstanding[b] = pltpu.async_copy(
            stg, out.at[bidx].at[pl.ds(srow0 + c * CH, CH)], so
        )

    for b in range(NBUF):
        if outstanding[b] is not None:
            outstanding[b].wait()


@jax.jit
def _run(table, types_all, pos_all, ids_all, hsn_all, hsn2d, lay):
    mesh = plsc.VectorSubcoreMesh(
        core_axis_name="c", subcore_axis_name="s", num_cores=NC, num_subcores=NS
    )
    f = functools.partial(
        pl.kernel,
        out_type=jax.ShapeDtypeStruct((B, S, F), jnp.float32),
        mesh=mesh,
        scratch_types=[
            pltpu.VMEM((CH, F), jnp.float32),
            pltpu.VMEM((CH, F), jnp.float32),
            pltpu.VMEM((CH, F), jnp.float32),
            pltpu.VMEM((T, Q), jnp.float32),
            pltpu.VMEM((RPW,), jnp.int32),
            pltpu.VMEM((RPW,), jnp.int32),
            pltpu.VMEM((RPW,), jnp.int32),
            pltpu.VMEM((RPW,), jnp.float32),
            pltpu.VMEM((RPW,), jnp.float32),
            pltpu.VMEM((RPW,), jnp.float32),
            pltpu.VMEM((L,), jnp.float32),
            pltpu.SemaphoreType.DMA,
            pltpu.SemaphoreType.DMA,
            pltpu.SemaphoreType.DMA,
        ],
        compiler_params=pltpu.CompilerParams(
            use_tc_tiling_on_sc=True, needs_layout_passes=False
        ),
    )(_body)
    return f(table, types_all, pos_all, ids_all, hsn_all, hsn2d, lay)


def kernel(input_ids, token_type_ids, positions, hidden_state_norms,
           layer_idx, token_type_table):
    types_all = token_type_ids.reshape(N)
    pos_all = positions.reshape(N)
    ids_all = input_ids.reshape(N)
    hsn_all = hidden_state_norms.reshape(N)
    hsn2d = hidden_state_norms.reshape(CH, N // CH)
    lay = jnp.zeros((L,), jnp.float32) + jnp.asarray(layer_idx, jnp.float32) / 100.0
    return _run(token_type_table, types_all, pos_all, ids_all, hsn_all, hsn2d, lay)


# speedup vs baseline: 3.9047x; 1.0269x over previous
import functools

import jax
import jax.numpy as jnp
from jax import lax
from jax.experimental import pallas as pl
from jax.experimental.pallas import tpu as pltpu
from jax.experimental.pallas import tpu_sc as plsc

B, S = 4, 8192
F = 1024
Q = 256
T = 10
N = B * S
NC, NS = 2, 16
NW = NC * NS
RPW = N // NW
CH = 32
NCH = RPW // CH
L = 16
NBUF = 3


def _body(table, types2d, pos2d, ids2d, hsn2d, lay,
          out,
          stg0, stg1, stg2, tloc, typesv, posi, idsi, posf, specf, hsnf, lay_v,
          so0, so1, so2):
    wid = lax.axis_index("s") * NC + lax.axis_index("c")
    row0w = wid * RPW
    bidx = row0w // S
    srow0 = row0w % S
    iota = lax.iota(jnp.int32, L)
    z16 = jnp.zeros((L,), jnp.float32)
    stgs = (stg0, stg1, stg2)
    sos = (so0, so1, so2)

    pltpu.sync_copy(hsn2d, stg0)

    def _mxrow(r, acc):
        for j in range(F // L):
            acc = jnp.maximum(acc, stg0[r, pl.ds(j * L, L)])
        return acc

    maxv = jnp.max(lax.fori_loop(0, CH, _mxrow, jnp.full((L,), -1.0, jnp.float32)))

    pltpu.sync_copy(table, tloc)
    pltpu.sync_copy(types2d.at[bidx].at[pl.ds(srow0, RPW)], typesv)
    pltpu.sync_copy(pos2d.at[bidx].at[pl.ds(srow0, RPW)], posi)
    pltpu.sync_copy(ids2d.at[bidx].at[pl.ds(srow0, RPW)], idsi)
    pltpu.sync_copy(hsn2d.at[wid], hsnf)
    pltpu.sync_copy(lay, lay_v)
    layv = lay_v[...]

    def _cols(j, carry):
        sl = pl.ds(j * L, L)
        posf[sl] = posi[sl].astype(jnp.float32) * (1.0 / S)
        specf[sl] = jnp.where(idsi[sl] < 5, 1.0, 0.0).astype(jnp.float32)
        hsnf[sl] = hsnf[sl] / maxv
        return carry

    lax.fori_loop(0, RPW // L, _cols, 0)

    def _zrow(r, carry):
        for stg in stgs:
            for k in range(Q, F, L):
                stg[r, pl.ds(k, L)] = z16
        return carry

    lax.fori_loop(0, CH, _zrow, 0)
    colL = jnp.full((L,), Q + 3, jnp.int32)
    for stg in stgs:
        for g in range(CH // L):
            rows = g * L + iota
            plsc.store_scatter(stg, [rows, colL], layv)

    cols = [iota + k * L for k in range(Q // L)]
    col0 = jnp.full((L,), Q + 0, jnp.int32)
    col1 = jnp.full((L,), Q + 1, jnp.int32)
    col2 = jnp.full((L,), Q + 2, jnp.int32)
    outstanding = [None] * NBUF

    for c in range(NCH):
        b = c % NBUF
        stg, so = stgs[b], sos[b]
        if outstanding[b] is not None:
            outstanding[b].wait()

        def _erow(r, carry, stg=stg, c=c):
            tfull = plsc.load_gather(typesv, [jnp.zeros((L,), jnp.int32) + (c * CH + r)])
            for k in range(Q // L):
                stg[r, pl.ds(k * L, L)] = plsc.load_gather(tloc, [tfull, cols[k]])
            return carry

        lax.fori_loop(0, CH, _erow, 0)

        for g in range(CH // L):
            off = c * CH + g * L
            rows = g * L + iota
            plsc.store_scatter(stg, [rows, col0], posf[pl.ds(off, L)])
            plsc.store_scatter(stg, [rows, col1], specf[pl.ds(off, L)])
            plsc.store_scatter(stg, [rows, col2], hsnf[pl.ds(off, L)])

        outstanding[b] = pltpu.async_copy(
            stg, out.at[bidx].at[pl.ds(srow0 + c * CH, CH)], so
        )

    for b in range(NBUF):
        if outstanding[b] is not None:
            outstanding[b].wait()


@jax.jit
def _run(table, types2d, pos2d, ids2d, hsn2d, lay):
    mesh = plsc.VectorSubcoreMesh(
        core_axis_name="c", subcore_axis_name="s", num_cores=NC, num_subcores=NS
    )
    f = functools.partial(
        pl.kernel,
        out_type=jax.ShapeDtypeStruct((B, S, F), jnp.float32),
        mesh=mesh,
        scratch_types=[
            pltpu.VMEM((CH, F), jnp.float32),
            pltpu.VMEM((CH, F), jnp.float32),
            pltpu.VMEM((CH, F), jnp.float32),
            pltpu.VMEM((T, Q), jnp.float32),
            pltpu.VMEM((RPW,), jnp.int32),
            pltpu.VMEM((RPW,), jnp.int32),
            pltpu.VMEM((RPW,), jnp.int32),
            pltpu.VMEM((RPW,), jnp.float32),
            pltpu.VMEM((RPW,), jnp.float32),
            pltpu.VMEM((RPW,), jnp.float32),
            pltpu.VMEM((L,), jnp.float32),
            pltpu.SemaphoreType.DMA,
            pltpu.SemaphoreType.DMA,
            pltpu.SemaphoreType.DMA,
        ],
        compiler_params=pltpu.CompilerParams(
            use_tc_tiling_on_sc=True, needs_layout_passes=False
        ),
    )(_body)
    return f(table, types2d, pos2d, ids2d, hsn2d, lay)


def kernel(input_ids, token_type_ids, positions, hidden_state_norms,
           layer_idx, token_type_table):
    hsn2d = hidden_state_norms.reshape(NW, RPW)
    lay = jnp.zeros((L,), jnp.float32) + jnp.asarray(layer_idx, jnp.float32) / 100.0
    return _run(token_type_table, token_type_ids, positions, input_ids, hsn2d, lay)


# speedup vs baseline: 3.9829x; 1.0200x over previous
import functools

import jax
import jax.numpy as jnp
from jax import lax
from jax.experimental import pallas as pl
from jax.experimental.pallas import tpu as pltpu
from jax.experimental.pallas import tpu_sc as plsc

B, S = 4, 8192
F = 1024
Q = 256
T = 10
N = B * S
NC, NS = 2, 16
NW = NC * NS
RPW = N // NW
CH = 64
NCH = RPW // CH
L = 16
NBUF = 3
HEAD = 384
TAILW = F - HEAD


def _body(table, types2d, pos2d, ids2d, hsn2d, lay,
          out,
          stg0, stg1, stg2, maxbuf, tloc, typesv, posi, idsi, posf, specf,
          hsnf, lay_v, zbuf,
          sh0, sh1, sh2, st0, st1, st2):
    wid = lax.axis_index("s") * NC + lax.axis_index("c")
    row0w = wid * RPW
    bidx = row0w // S
    srow0 = row0w % S
    iota = lax.iota(jnp.int32, L)
    z16 = jnp.zeros((L,), jnp.float32)
    stgs = (stg0, stg1, stg2)
    shs = (sh0, sh1, sh2)
    sts = (st0, st1, st2)

    pltpu.sync_copy(hsn2d, maxbuf)

    def _mxrow(r, acc):
        for j in range(F // L):
            acc = jnp.maximum(acc, maxbuf[r, pl.ds(j * L, L)])
        return acc

    maxv = jnp.max(lax.fori_loop(0, NW, _mxrow, jnp.full((L,), -1.0, jnp.float32)))

    def _zmax(r, carry):
        for j in range(F // L):
            maxbuf[r, pl.ds(j * L, L)] = z16
        return carry

    lax.fori_loop(0, NW, _zmax, 0)
    pltpu.sync_copy(maxbuf.at[pl.ds(0, NW), pl.ds(0, TAILW)], zbuf.at[pl.ds(0, NW)])
    pltpu.sync_copy(maxbuf.at[pl.ds(0, NW), pl.ds(0, TAILW)], zbuf.at[pl.ds(NW, NW)])
    plsc.subcore_barrier()

    pltpu.sync_copy(table, tloc)
    pltpu.sync_copy(types2d.at[bidx].at[pl.ds(srow0, RPW)], typesv)
    pltpu.sync_copy(pos2d.at[bidx].at[pl.ds(srow0, RPW)], posi)
    pltpu.sync_copy(ids2d.at[bidx].at[pl.ds(srow0, RPW)], idsi)
    pltpu.sync_copy(hsn2d.at[wid], hsnf)
    pltpu.sync_copy(lay, lay_v)
    layv = lay_v[...]

    def _cols(j, carry):
        sl = pl.ds(j * L, L)
        posf[sl] = posi[sl].astype(jnp.float32) * (1.0 / S)
        specf[sl] = jnp.where(idsi[sl] < 5, 1.0, 0.0).astype(jnp.float32)
        hsnf[sl] = hsnf[sl] / maxv
        return carry

    lax.fori_loop(0, RPW // L, _cols, 0)

    def _zrow(r, carry):
        for stg in stgs:
            for k in range(Q, HEAD, L):
                stg[r, pl.ds(k, L)] = z16
        return carry

    lax.fori_loop(0, CH, _zrow, 0)
    colL = jnp.full((L,), Q + 3, jnp.int32)
    for stg in stgs:
        for g in range(CH // L):
            rows = g * L + iota
            plsc.store_scatter(stg, [rows, colL], layv)

    cols = [iota + k * L for k in range(Q // L)]
    col0 = jnp.full((L,), Q + 0, jnp.int32)
    col1 = jnp.full((L,), Q + 1, jnp.int32)
    col2 = jnp.full((L,), Q + 2, jnp.int32)
    out_h = [None] * NBUF
    out_t = [None] * NBUF

    for c in range(NCH):
        b = c % NBUF
        stg = stgs[b]
        rows_sl = pl.ds(srow0 + c * CH, CH)

        if out_t[b] is not None:
            out_t[b].wait()
        out_t[b] = pltpu.async_copy(
            zbuf, out.at[bidx, rows_sl, pl.ds(HEAD, TAILW)], sts[b]
        )

        if out_h[b] is not None:
            out_h[b].wait()

        def _erow(r, carry, stg=stg, c=c):
            tfull = plsc.load_gather(typesv, [jnp.zeros((L,), jnp.int32) + (c * CH + r)])
            for k in range(Q // L):
                stg[r, pl.ds(k * L, L)] = plsc.load_gather(tloc, [tfull, cols[k]])
            return carry

        lax.fori_loop(0, CH, _erow, 0)

        for g in range(CH // L):
            off = c * CH + g * L
            rows = g * L + iota
            plsc.store_scatter(stg, [rows, col0], posf[pl.ds(off, L)])
            plsc.store_scatter(stg, [rows, col1], specf[pl.ds(off, L)])
            plsc.store_scatter(stg, [rows, col2], hsnf[pl.ds(off, L)])

        out_h[b] = pltpu.async_copy(
            stg, out.at[bidx, rows_sl, pl.ds(0, HEAD)], shs[b]
        )

    for b in range(NBUF):
        if out_h[b] is not None:
            out_h[b].wait()
        if out_t[b] is not None:
            out_t[b].wait()


@jax.jit
def _run(table, types2d, pos2d, ids2d, hsn2d, lay):
    mesh = plsc.VectorSubcoreMesh(
        core_axis_name="c", subcore_axis_name="s", num_cores=NC, num_subcores=NS
    )
    f = functools.partial(
        pl.kernel,
        out_type=jax.ShapeDtypeStruct((B, S, F), jnp.float32),
        mesh=mesh,
        scratch_types=[
            pltpu.VMEM((CH, HEAD), jnp.float32),
            pltpu.VMEM((CH, HEAD), jnp.float32),
            pltpu.VMEM((CH, HEAD), jnp.float32),
            pltpu.VMEM((NW, RPW), jnp.float32),
            pltpu.VMEM((T, Q), jnp.float32),
            pltpu.VMEM((RPW,), jnp.int32),
            pltpu.VMEM((RPW,), jnp.int32),
            pltpu.VMEM((RPW,), jnp.int32),
            pltpu.VMEM((RPW,), jnp.float32),
            pltpu.VMEM((RPW,), jnp.float32),
            pltpu.VMEM((RPW,), jnp.float32),
            pltpu.VMEM((L,), jnp.float32),
            pltpu.VMEM_SHARED((CH, TAILW), jnp.float32),
            pltpu.SemaphoreType.DMA,
            pltpu.SemaphoreType.DMA,
            pltpu.SemaphoreType.DMA,
            pltpu.SemaphoreType.DMA,
            pltpu.SemaphoreType.DMA,
            pltpu.SemaphoreType.DMA,
        ],
        compiler_params=pltpu.CompilerParams(
            use_tc_tiling_on_sc=True, needs_layout_passes=False
        ),
    )(_body)
    return f(table, types2d, pos2d, ids2d, hsn2d, lay)


def kernel(input_ids, token_type_ids, positions, hidden_state_norms,
           layer_idx, token_type_table):
    hsn2d = hidden_state_norms.reshape(NW, RPW)
    lay = jnp.zeros((L,), jnp.float32) + jnp.asarray(layer_idx, jnp.float32) / 100.0
    return _run(token_type_table, token_type_ids, positions, input_ids, hsn2d, lay)
